# baseline (device time: 15311 ns/iter reference)
import jax
import jax.numpy as jnp
from jax import lax
from jax.experimental import pallas as pl
from jax.experimental.pallas import tpu as pltpu

N_DEV = 8
K = 8
OFFSETS = (1, 3, 4)
N_ROUNDS = 3


def _topk_desc(a, k, axes):
    cols = []
    for _ in range(k):
        m = jnp.max(a, axis=axes, keepdims=True)
        cols.append(m)
        a = jnp.where(a == m, -jnp.inf, a)
    return cols


def kernel(x):
    m_rows, n = x.shape

    def body(x_ref, out_ref, send_buf, recv_buf, send_sems, recv_sems,
             ready_sems):
        my = lax.axis_index("i")
        partners = [my ^ o for o in OFFSETS]

        barrier_sem = pltpu.get_barrier_semaphore()
        pl.semaphore_signal(
            barrier_sem, inc=1,
            device_id=(partners[0],), device_id_type=pl.DeviceIdType.MESH,
        )
        for r in (1, 2):
            pl.semaphore_signal(
                ready_sems.at[r - 1], inc=1,
                device_id=(partners[r],), device_id_type=pl.DeviceIdType.MESH,
            )

        xb = x_ref[:, :].astype(jnp.bfloat16)
        cols = _topk_desc(xb, K, axes=1)
        cur = jnp.concatenate(cols, axis=1)

        rdmas = []
        for r in range(N_ROUNDS):
            if r == 0:
                pl.semaphore_wait(barrier_sem, 1)
            else:
                pl.semaphore_wait(ready_sems.at[r - 1], 1)
            send_buf[r] = cur
            rdma = pltpu.make_async_remote_copy(
                src_ref=send_buf.at[r],
                dst_ref=recv_buf.at[r],
                send_sem=send_sems.at[r],
                recv_sem=recv_sems.at[r],
                device_id=(partners[r],),
                device_id_type=pl.DeviceIdType.MESH,
            )
            rdma.start()
            rdma.wait_recv()
            rdmas.append(rdma)
            both = jnp.concatenate([cur, recv_buf[r]], axis=1)
            cur = jnp.concatenate(_topk_desc(both, K, axes=1), axis=1)

        out_ref[:, :] = cur.astype(jnp.float32)

        for rdma in rdmas:
            rdma.wait_send()

    return pl.pallas_call(
        body,
        out_shape=jax.ShapeDtypeStruct((m_rows, K), jnp.float32),
        in_specs=[pl.BlockSpec(memory_space=pltpu.VMEM)],
        out_specs=pl.BlockSpec(memory_space=pltpu.VMEM),
        scratch_shapes=[
            pltpu.VMEM((N_ROUNDS, m_rows, K), jnp.bfloat16),
            pltpu.VMEM((N_ROUNDS, m_rows, K), jnp.bfloat16),
            pltpu.SemaphoreType.DMA((N_ROUNDS,)),
            pltpu.SemaphoreType.DMA((N_ROUNDS,)),
            pltpu.SemaphoreType.REGULAR((2,)),
        ],
        compiler_params=pltpu.CompilerParams(collective_id=0),
    )(x)


# device time: 12625 ns/iter; 1.2128x vs baseline; 1.2128x over previous
import jax
import jax.numpy as jnp
from jax import lax
from jax.experimental import pallas as pl
from jax.experimental.pallas import tpu as pltpu

N_DEV = 8
K = 8


def _topk_desc(a, k, axes):
    cols = []
    for _ in range(k):
        m = jnp.max(a, axis=axes, keepdims=True)
        cols.append(m)
        a = jnp.where(a == m, -jnp.inf, a)
    return cols


def kernel(x):
    m_rows, n = x.shape

    def body(x_ref, out_ref, allc_ref, send_sems, recv_sems, credit_sems):
        my = lax.axis_index("i")

        barrier_sem = pltpu.get_barrier_semaphore()
        pl.semaphore_signal(
            barrier_sem, inc=1,
            device_id=(my ^ 1,), device_id_type=pl.DeviceIdType.MESH,
        )
        for o in range(2, N_DEV):
            pl.semaphore_signal(
                credit_sems.at[o - 2], inc=1,
                device_id=(my ^ o,), device_id_type=pl.DeviceIdType.MESH,
            )

        xb = x_ref[:, :].astype(jnp.bfloat16)
        cols = _topk_desc(xb, K, axes=1)
        allc_ref[0] = jnp.concatenate(cols, axis=1)

        rdmas = []
        for o in range(1, N_DEV):
            if o == 1:
                pl.semaphore_wait(barrier_sem, 1)
            else:
                pl.semaphore_wait(credit_sems.at[o - 2], 1)
            rdma = pltpu.make_async_remote_copy(
                src_ref=allc_ref.at[0],
                dst_ref=allc_ref.at[o],
                send_sem=send_sems.at[o - 1],
                recv_sem=recv_sems.at[o - 1],
                device_id=(my ^ o,),
                device_id_type=pl.DeviceIdType.MESH,
            )
            rdma.start()
            rdmas.append(rdma)
        for rdma in rdmas:
            rdma.wait_recv()

        allc = allc_ref[:, :, :]
        outs = _topk_desc(allc, K, axes=(0, 2))
        out_ref[:, :] = jnp.concatenate(
            [o.reshape(m_rows, 1) for o in outs], axis=1
        ).astype(jnp.float32)

        for rdma in rdmas:
            rdma.wait_send()

    return pl.pallas_call(
        body,
        out_shape=jax.ShapeDtypeStruct((m_rows, K), jnp.float32),
        in_specs=[pl.BlockSpec(memory_space=pltpu.VMEM)],
        out_specs=pl.BlockSpec(memory_space=pltpu.VMEM),
        scratch_shapes=[
            pltpu.VMEM((N_DEV, m_rows, K), jnp.bfloat16),
            pltpu.SemaphoreType.DMA((N_DEV - 1,)),
            pltpu.SemaphoreType.DMA((N_DEV - 1,)),
            pltpu.SemaphoreType.REGULAR((N_DEV - 2,)),
        ],
        compiler_params=pltpu.CompilerParams(collective_id=0),
    )(x)


# device time: 12288 ns/iter; 1.2460x vs baseline; 1.0274x over previous
import jax
import jax.numpy as jnp
from jax import lax
from jax.experimental import pallas as pl
from jax.experimental.pallas import tpu as pltpu

N_DEV = 8
K = 8
SEND_ORDER = (1, 3, 4, 2, 5, 7, 6)
NEAR_SLOTS = (1, 2, 3, 4, 5)


def _topk_desc(a, k, axes):
    cols = []
    for _ in range(k):
        m = jnp.max(a, axis=axes, keepdims=True)
        cols.append(m)
        a = jnp.where(a == m, -jnp.inf, a)
    return cols


def kernel(x):
    m_rows, n = x.shape

    def body(x_hbm, out_ref, xv_ref, allc_ref, send_sems, recv_sems,
             credit_sems, copy_sem):
        my = lax.axis_index("i")

        barrier_sem = pltpu.get_barrier_semaphore()
        pl.semaphore_signal(
            barrier_sem, inc=1,
            device_id=(my ^ 1,), device_id_type=pl.DeviceIdType.MESH,
        )
        for o in range(2, N_DEV):
            pl.semaphore_signal(
                credit_sems.at[o - 2], inc=1,
                device_id=(my ^ o,), device_id_type=pl.DeviceIdType.MESH,
            )

        cp = pltpu.make_async_copy(x_hbm, xv_ref, copy_sem)
        cp.start()
        cp.wait()
        xb = xv_ref[:, :].astype(jnp.bfloat16)
        cols = _topk_desc(xb, K, axes=1)
        allc_ref[0] = jnp.concatenate(cols, axis=1)

        rdmas = {}
        for o in SEND_ORDER:
            if o == 1:
                pl.semaphore_wait(barrier_sem, 1)
            else:
                pl.semaphore_wait(credit_sems.at[o - 2], 1)
            rdma = pltpu.make_async_remote_copy(
                src_ref=allc_ref.at[0],
                dst_ref=allc_ref.at[o],
                send_sem=send_sems.at[o - 1],
                recv_sem=recv_sems.at[o - 1],
                device_id=(my ^ o,),
                device_id_type=pl.DeviceIdType.MESH,
            )
            rdma.start()
            rdmas[o] = rdma

        for s in NEAR_SLOTS:
            rdmas[s].wait_recv()
        near = allc_ref[pl.ds(0, 6)]
        pre = _topk_desc(near, K, axes=(0, 2))

        rdmas[7].wait_recv()
        rdmas[6].wait_recv()
        tail = jnp.concatenate(
            [o.reshape(m_rows, 1) for o in pre]
            + [allc_ref[6], allc_ref[7]],
            axis=1,
        )
        outs = _topk_desc(tail, K, axes=1)
        out_ref[:, :] = jnp.concatenate(outs, axis=1).astype(jnp.float32)

        for o in SEND_ORDER:
            rdmas[o].wait_send()

    return pl.pallas_call(
        body,
        out_shape=jax.ShapeDtypeStruct((m_rows, K), jnp.float32),
        in_specs=[pl.BlockSpec(memory_space=pltpu.MemorySpace.HBM)],
        out_specs=pl.BlockSpec(memory_space=pltpu.VMEM),
        scratch_shapes=[
            pltpu.VMEM((m_rows, n), jnp.float32),
            pltpu.VMEM((N_DEV, m_rows, K), jnp.bfloat16),
            pltpu.SemaphoreType.DMA((N_DEV - 1,)),
            pltpu.SemaphoreType.DMA((N_DEV - 1,)),
            pltpu.SemaphoreType.REGULAR((N_DEV - 2,)),
            pltpu.SemaphoreType.DMA,
        ],
        compiler_params=pltpu.CompilerParams(collective_id=0),
    )(x)
